# Initial kernel scaffold; baseline (speedup 1.0000x reference)
import jax
import jax.numpy as jnp
from jax.experimental import pallas as pl


def kernel(preds, img_size):
    raise NotImplementedError("write your pallas kernel here")


# speedup vs baseline: 8.4151x; 8.4151x over previous
import jax
import jax.numpy as jnp
from jax.experimental import pallas as pl
from jax.experimental.pallas import tpu as pltpu

_CONF = 0.25
_IOU = 0.45
_MAX_DET = 300
_PRE_K = 1000
_MAX_WH = 7680.0

_B = 8
_NC = 80
_NANCH = 8400
_NPAD = 8448
_K = 1024
_BLK = 128
_CCH = 256
_OUTROWS = 384


def _decode_kernel(img_ref, p_ref, conf_ref, cls_ref, x1_ref, y1_ref, x2_ref, y2_ref):
    img = img_ref[0, 0]
    p = p_ref[0]
    cx = p[0:1] * img
    cy = p[1:2] * img
    w = p[2:3] * img
    h = p[3:4] * img
    x1_ref[0] = cx - w / 2.0
    y1_ref[0] = cy - h / 2.0
    x2_ref[0] = cx + w / 2.0
    y2_ref[0] = cy + h / 2.0
    cls_scores = p[4:4 + _NC]
    conf = jnp.max(cls_scores, axis=0, keepdims=True)
    conf_ref[0] = conf
    row = jax.lax.broadcasted_iota(jnp.int32, (_NC, _NPAD), 0).astype(jnp.float32)
    cls_ref[0] = jnp.min(jnp.where(cls_scores == conf, row, 1e9), axis=0, keepdims=True)


def _nms_kernel(img_ref, sc_ref, x1_ref, y1_ref, x2_ref, y2_ref, cls_ref,
                out_ref, over_ref, diag_ref):
    img = img_ref[0, 0]
    scores = sc_ref[...]
    cls = cls_ref[...]
    off = cls * _MAX_WH
    ox1 = x1_ref[...] + off
    oy1 = y1_ref[...] + off
    ox2 = x2_ref[...] + off
    oy2 = y2_ref[...] + off
    area = (ox2 - ox1) * (oy2 - oy1)

    keep = (scores > _CONF).astype(jnp.float32)
    col_iota = jax.lax.broadcasted_iota(jnp.int32, (_B, _K), 1)
    lane_iota = jax.lax.broadcasted_iota(jnp.int32, (_B, _BLK), 1)

    keep_parts = []
    for blk in range(_K // _BLK):
        s0 = blk * _BLK
        rx1 = ox1[:, s0:s0 + _BLK][:, :, None]
        ry1 = oy1[:, s0:s0 + _BLK][:, :, None]
        rx2 = ox2[:, s0:s0 + _BLK][:, :, None]
        ry2 = oy2[:, s0:s0 + _BLK][:, :, None]
        rarea = area[:, s0:s0 + _BLK][:, :, None]
        for cb in range(blk * _BLK // _CCH, _K // _CCH):
            c0 = cb * _CCH
            cx1 = ox1[:, None, c0:c0 + _CCH]
            cy1 = oy1[:, None, c0:c0 + _CCH]
            cx2 = ox2[:, None, c0:c0 + _CCH]
            cy2 = oy2[:, None, c0:c0 + _CCH]
            carea = area[:, None, c0:c0 + _CCH]
            ltx = jnp.maximum(rx1, cx1)
            lty = jnp.maximum(ry1, cy1)
            rbx = jnp.minimum(rx2, cx2)
            rby = jnp.minimum(ry2, cy2)
            iw = jnp.clip(rbx - ltx, 0.0, None)
            ih = jnp.clip(rby - lty, 0.0, None)
            inter = iw * ih
            iou = inter / (rarea + carea - inter + 1e-7)
            over_ref[:, :, c0:c0 + _CCH] = (iou > _IOU).astype(jnp.float32)

        tx1 = ox1[:, s0:s0 + _BLK].T[:, :, None]
        ty1 = oy1[:, s0:s0 + _BLK].T[:, :, None]
        tx2 = ox2[:, s0:s0 + _BLK].T[:, :, None]
        ty2 = oy2[:, s0:s0 + _BLK].T[:, :, None]
        tarea = area[:, s0:s0 + _BLK].T[:, :, None]
        dx1 = ox1[:, s0:s0 + _BLK][None]
        dy1 = oy1[:, s0:s0 + _BLK][None]
        dx2 = ox2[:, s0:s0 + _BLK][None]
        dy2 = oy2[:, s0:s0 + _BLK][None]
        darea = area[:, s0:s0 + _BLK][None]
        dltx = jnp.maximum(tx1, dx1)
        dlty = jnp.maximum(ty1, dy1)
        drbx = jnp.minimum(tx2, dx2)
        drby = jnp.minimum(ty2, dy2)
        diw = jnp.clip(drbx - dltx, 0.0, None)
        dih = jnp.clip(drby - dlty, 0.0, None)
        dinter = diw * dih
        diou = dinter / (tarea + darea - dinter + 1e-7)
        diag_ref[...] = (diou > _IOU).astype(jnp.float32)

        kb0 = keep[:, s0:s0 + _BLK]

        def body(i, kb):
            rowi = diag_ref[pl.ds(i, 1)][0]
            onehot = (lane_iota == i).astype(jnp.float32)
            ki = jnp.sum(kb * onehot, axis=1, keepdims=True)
            gt = (lane_iota > i).astype(jnp.float32)
            return kb * (1.0 - rowi * gt * ki)

        kb = jax.lax.fori_loop(0, _BLK, body, kb0)
        keep_parts.append(kb)

        w = over_ref[...] * kb[:, :, None]
        sup_cols = jnp.max(w, axis=1)
        future = (col_iota >= (s0 + _BLK)).astype(jnp.float32)
        keep = keep * (1.0 - sup_cols * future)

    keepf = jnp.concatenate(keep_parts, axis=1)

    c = keepf
    sh = 1
    while sh < _K:
        c = c + jnp.concatenate(
            [jnp.zeros((_B, sh), jnp.float32), c[:, :-sh]], axis=1)
        sh *= 2
    pos = c - 1.0

    n1 = x1_ref[...] / img
    n2 = y1_ref[...] / img
    n3 = x2_ref[...] / img
    n4 = y2_ref[...] / img
    rows_iota = jax.lax.broadcasted_iota(jnp.int32, (_OUTROWS, _K), 0).astype(jnp.float32)
    for b in range(_B):
        sel = (rows_iota == pos[b:b + 1]) & (keepf[b:b + 1] > 0.0)
        pb = jnp.where(sel, 1.0, 0.0)
        cols = []
        for ch in (cls, scores, n1, n2, n3, n4):
            cols.append(jnp.sum(pb * ch[b:b + 1], axis=1, keepdims=True))
        cols.append(jnp.zeros((_OUTROWS, 2), jnp.float32))
        out_ref[b] = jnp.concatenate(cols, axis=1)


def kernel(preds, img_size):
    img = jnp.asarray(img_size, jnp.float32).reshape(1, 1)
    p = jnp.pad(preds, ((0, 0), (0, 0), (0, _NPAD - preds.shape[2])))

    plane = jax.ShapeDtypeStruct((_B, 1, _NPAD), jnp.float32)
    smem = pl.BlockSpec(memory_space=pltpu.SMEM)
    conf, clsf, x1, y1, x2, y2 = pl.pallas_call(
        _decode_kernel,
        grid=(_B,),
        in_specs=[
            pl.BlockSpec((1, 1), lambda b: (0, 0), memory_space=pltpu.SMEM),
            pl.BlockSpec((1, 84, _NPAD), lambda b: (b, 0, 0)),
        ],
        out_specs=[pl.BlockSpec((1, 1, _NPAD), lambda b: (b, 0, 0))] * 6,
        out_shape=[plane] * 6,
    )(img, p)

    conf2 = conf[:, 0, :_NANCH]
    masked = jnp.where(conf2 > _CONF, conf2, -1.0)
    top_s, top_i = jax.lax.top_k(masked, _PRE_K)
    top_s = jnp.pad(top_s, ((0, 0), (0, _K - _PRE_K)), constant_values=-1.0)
    top_i = jnp.pad(top_i, ((0, 0), (0, _K - _PRE_K)))

    def g(a):
        return jnp.take_along_axis(a[:, 0, :], top_i, axis=1)

    out = pl.pallas_call(
        _nms_kernel,
        in_specs=[smem] + [pl.BlockSpec() for _ in range(6)],
        out_shape=jax.ShapeDtypeStruct((_B, _OUTROWS, 8), jnp.float32),
        scratch_shapes=[pltpu.VMEM((_B, _BLK, _K), jnp.float32),
                        pltpu.VMEM((_BLK, _B, _BLK), jnp.float32)],
    )(img, top_s, g(x1), g(y1), g(x2), g(y2), g(clsf))
    return out[:, :_MAX_DET, :6]


# speedup vs baseline: 9.2222x; 1.0959x over previous
import jax
import jax.numpy as jnp
from jax.experimental import pallas as pl
from jax.experimental.pallas import tpu as pltpu

_CONF = 0.25
_IOU = 0.45
_MAX_DET = 300
_PRE_K = 1000
_MAX_WH = 7680.0

_B = 8
_NC = 80
_NANCH = 8400
_NPAD = 9216
_K = 1024
_BLK = 128
_CCH = 256
_OUTROWS = 384


def _decode_kernel(img_ref, p_ref, conf_ref, cls_ref, x1_ref, y1_ref, x2_ref, y2_ref):
    img = img_ref[0, 0]
    p = p_ref[0]
    cx = p[0:1] * img
    cy = p[1:2] * img
    w = p[2:3] * img
    h = p[3:4] * img
    x1_ref[0] = cx - w / 2.0
    y1_ref[0] = cy - h / 2.0
    x2_ref[0] = cx + w / 2.0
    y2_ref[0] = cy + h / 2.0
    cls_scores = p[4:4 + _NC]
    conf = jnp.max(cls_scores, axis=0, keepdims=True)
    conf_ref[0] = conf
    row = jax.lax.broadcasted_iota(jnp.int32, (_NC, _NPAD), 0).astype(jnp.float32)
    cls_ref[0] = jnp.min(jnp.where(cls_scores == conf, row, 1e9), axis=0, keepdims=True)


def _gt(sa, ia, sb, ib):
    return (sa > sb) | ((sa == sb) & (ia < ib))


def _sort_kernel(conf_ref, s_ref, i_ref):
    conf = conf_ref[...]
    masked = jnp.where(conf > _CONF, conf, -1.0)
    idxf = jax.lax.broadcasted_iota(jnp.int32, (_B, _NPAD), 1).astype(jnp.float32)
    lane = jax.lax.broadcasted_iota(jnp.int32, (_B, _K), 1)

    def xor_swap(x, j):
        bj0 = (lane & j) == 0
        return jnp.where(bj0, jnp.roll(x, -j, axis=1), jnp.roll(x, j, axis=1))

    def rev(x):
        j = _K // 2
        while j >= 1:
            x = xor_swap(x, j)
            j //= 2
        return x

    def cx(s, idx, j, wg):
        bj0 = (lane & j) == 0
        ps = jnp.where(bj0, jnp.roll(s, -j, axis=1), jnp.roll(s, j, axis=1))
        pi = jnp.where(bj0, jnp.roll(idx, -j, axis=1), jnp.roll(idx, j, axis=1))
        win = _gt(s, idx, ps, pi)
        keep_self = wg == win
        return jnp.where(keep_self, s, ps), jnp.where(keep_self, idx, pi)

    def sort_desc(s, idx):
        k = 2
        while k <= _K:
            j = k // 2
            while j >= 1:
                bj0 = (lane & j) == 0
                wg = bj0 == ((lane & k) == 0)
                s, idx = cx(s, idx, j, wg)
                j //= 2
            k *= 2
        return s, idx

    parts = []
    for c in range(_NPAD // _K):
        sl = slice(c * _K, (c + 1) * _K)
        parts.append(sort_desc(masked[:, sl], idxf[:, sl]))

    ms, mi = parts[0]
    for c in range(1, len(parts)):
        bs, bi = parts[c]
        rbs = rev(bs)
        rbi = rev(bi)
        win = _gt(ms, mi, rbs, rbi)
        s = jnp.where(win, ms, rbs)
        idx = jnp.where(win, mi, rbi)
        j = _K // 2
        while j >= 1:
            s, idx = cx(s, idx, j, (lane & j) == 0)
            j //= 2
        ms, mi = s, idx

    s_ref[...] = jnp.where(lane < _PRE_K, ms, -1.0)
    i_ref[...] = mi


def _nms_kernel(img_ref, sc_ref, x1_ref, y1_ref, x2_ref, y2_ref, cls_ref,
                out_ref, over_ref, diag_ref):
    img = img_ref[0, 0]
    scores = sc_ref[...]
    cls = cls_ref[...]
    off = cls * _MAX_WH
    ox1 = x1_ref[...] + off
    oy1 = y1_ref[...] + off
    ox2 = x2_ref[...] + off
    oy2 = y2_ref[...] + off
    area = (ox2 - ox1) * (oy2 - oy1)

    keep = (scores > _CONF).astype(jnp.float32)
    col_iota = jax.lax.broadcasted_iota(jnp.int32, (_B, _K), 1)
    lane_iota = jax.lax.broadcasted_iota(jnp.int32, (_B, _BLK), 1)

    keep_parts = []
    for blk in range(_K // _BLK):
        s0 = blk * _BLK
        rx1 = ox1[:, s0:s0 + _BLK][:, :, None]
        ry1 = oy1[:, s0:s0 + _BLK][:, :, None]
        rx2 = ox2[:, s0:s0 + _BLK][:, :, None]
        ry2 = oy2[:, s0:s0 + _BLK][:, :, None]
        rarea = area[:, s0:s0 + _BLK][:, :, None]
        for cb in range(blk * _BLK // _CCH, _K // _CCH):
            c0 = cb * _CCH
            cx1 = ox1[:, None, c0:c0 + _CCH]
            cy1 = oy1[:, None, c0:c0 + _CCH]
            cx2 = ox2[:, None, c0:c0 + _CCH]
            cy2 = oy2[:, None, c0:c0 + _CCH]
            carea = area[:, None, c0:c0 + _CCH]
            ltx = jnp.maximum(rx1, cx1)
            lty = jnp.maximum(ry1, cy1)
            rbx = jnp.minimum(rx2, cx2)
            rby = jnp.minimum(ry2, cy2)
            iw = jnp.clip(rbx - ltx, 0.0, None)
            ih = jnp.clip(rby - lty, 0.0, None)
            inter = iw * ih
            iou = inter / (rarea + carea - inter + 1e-7)
            over_ref[:, :, c0:c0 + _CCH] = (iou > _IOU).astype(jnp.float32)

        tx1 = ox1[:, s0:s0 + _BLK].T[:, :, None]
        ty1 = oy1[:, s0:s0 + _BLK].T[:, :, None]
        tx2 = ox2[:, s0:s0 + _BLK].T[:, :, None]
        ty2 = oy2[:, s0:s0 + _BLK].T[:, :, None]
        tarea = area[:, s0:s0 + _BLK].T[:, :, None]
        dx1 = ox1[:, s0:s0 + _BLK][None]
        dy1 = oy1[:, s0:s0 + _BLK][None]
        dx2 = ox2[:, s0:s0 + _BLK][None]
        dy2 = oy2[:, s0:s0 + _BLK][None]
        darea = area[:, s0:s0 + _BLK][None]
        dltx = jnp.maximum(tx1, dx1)
        dlty = jnp.maximum(ty1, dy1)
        drbx = jnp.minimum(tx2, dx2)
        drby = jnp.minimum(ty2, dy2)
        diw = jnp.clip(drbx - dltx, 0.0, None)
        dih = jnp.clip(drby - dlty, 0.0, None)
        dinter = diw * dih
        diou = dinter / (tarea + darea - dinter + 1e-7)
        diag_ref[...] = (diou > _IOU).astype(jnp.float32)

        kb0 = keep[:, s0:s0 + _BLK]

        def body(i, kb):
            rowi = diag_ref[pl.ds(i, 1)][0]
            onehot = (lane_iota == i).astype(jnp.float32)
            ki = jnp.sum(kb * onehot, axis=1, keepdims=True)
            gt = (lane_iota > i).astype(jnp.float32)
            return kb * (1.0 - rowi * gt * ki)

        kb = jax.lax.fori_loop(0, _BLK, body, kb0)
        keep_parts.append(kb)

        w = over_ref[...] * kb[:, :, None]
        sup_cols = jnp.max(w, axis=1)
        future = (col_iota >= (s0 + _BLK)).astype(jnp.float32)
        keep = keep * (1.0 - sup_cols * future)

    keepf = jnp.concatenate(keep_parts, axis=1)

    c = keepf
    sh = 1
    while sh < _K:
        c = c + jnp.concatenate(
            [jnp.zeros((_B, sh), jnp.float32), c[:, :-sh]], axis=1)
        sh *= 2
    pos = c - 1.0

    n1 = x1_ref[...] / img
    n2 = y1_ref[...] / img
    n3 = x2_ref[...] / img
    n4 = y2_ref[...] / img
    rows_iota = jax.lax.broadcasted_iota(jnp.int32, (_OUTROWS, _K), 0).astype(jnp.float32)
    for b in range(_B):
        sel = (rows_iota == pos[b:b + 1]) & (keepf[b:b + 1] > 0.0)
        pb = jnp.where(sel, 1.0, 0.0)
        cols = []
        for ch in (cls, scores, n1, n2, n3, n4):
            cols.append(jnp.sum(pb * ch[b:b + 1], axis=1, keepdims=True))
        cols.append(jnp.zeros((_OUTROWS, 2), jnp.float32))
        out_ref[b] = jnp.concatenate(cols, axis=1)


def kernel(preds, img_size):
    img = jnp.asarray(img_size, jnp.float32).reshape(1, 1)
    p = jnp.pad(preds, ((0, 0), (0, 0), (0, _NPAD - preds.shape[2])))

    plane = jax.ShapeDtypeStruct((_B, 1, _NPAD), jnp.float32)
    smem = pl.BlockSpec(memory_space=pltpu.SMEM)
    conf, clsf, x1, y1, x2, y2 = pl.pallas_call(
        _decode_kernel,
        grid=(_B,),
        in_specs=[
            pl.BlockSpec((1, 1), lambda b: (0, 0), memory_space=pltpu.SMEM),
            pl.BlockSpec((1, 84, _NPAD), lambda b: (b, 0, 0)),
        ],
        out_specs=[pl.BlockSpec((1, 1, _NPAD), lambda b: (b, 0, 0))] * 6,
        out_shape=[plane] * 6,
    )(img, p)

    top_s, top_if = pl.pallas_call(
        _sort_kernel,
        out_shape=[jax.ShapeDtypeStruct((_B, _K), jnp.float32)] * 2,
    )(conf[:, 0, :])
    top_i = top_if.astype(jnp.int32)

    def g(a):
        return jnp.take_along_axis(a[:, 0, :], top_i, axis=1, mode="clip")

    out = pl.pallas_call(
        _nms_kernel,
        in_specs=[smem] + [pl.BlockSpec() for _ in range(6)],
        out_shape=jax.ShapeDtypeStruct((_B, _OUTROWS, 8), jnp.float32),
        scratch_shapes=[pltpu.VMEM((_B, _BLK, _K), jnp.float32),
                        pltpu.VMEM((_BLK, _B, _BLK), jnp.float32)],
    )(img, top_s, g(x1), g(y1), g(x2), g(y2), g(clsf))
    return out[:, :_MAX_DET, :6]
